# baseline (speedup 1.0000x reference)
import functools

import jax
import jax.numpy as jnp
import numpy as np
from jax import lax
from jax.experimental import pallas as pl
from jax.experimental.pallas import tpu as pltpu
from jax.experimental.pallas import tpu_sc as plsc

_LANES = 16


def _build_sc_call(R, D, T, A, P, AP):
    NC, NS = 2, 16
    NW = NC * NS
    CH = 40
    CPA = (P * T) // CH
    NV = D // _LANES

    assert T % CH == 0 and (P * T) % CH == 0 and CH % 8 == 0

    mesh = plsc.VectorSubcoreMesh(core_axis_name="c", subcore_axis_name="s")

    @functools.partial(
        pl.kernel,
        out_type=jax.ShapeDtypeStruct((R, D), jnp.float32),
        mesh=mesh,
        scratch_types=[
            pltpu.VMEM((T, D), jnp.float32),
            pltpu.VMEM((AP, D), jnp.float32),
            pltpu.VMEM((AP, _LANES), jnp.int32),
            pltpu.VMEM((NW, _LANES), jnp.int32),
            pltpu.VMEM((CH, D), jnp.float32),
            pltpu.VMEM((CH, D), jnp.float32),
            pltpu.VMEM((CH, D), jnp.float32),
            pltpu.VMEM((CH, D), jnp.float32),
            pltpu.SemaphoreType.DMA,
            pltpu.SemaphoreType.DMA,
            pltpu.SemaphoreType.DMA,
            pltpu.SemaphoreType.DMA,
            pltpu.SemaphoreType.DMA,
            pltpu.SemaphoreType.DMA,
            pltpu.SemaphoreType.DMA,
            pltpu.SemaphoreType.DMA,
        ],
    )
    def sc_add(x_hbm, pe_hbm, ie_hbm, idx_hbm, winit_hbm, out_hbm,
               pe_b, ie_b, idx_b, wi_b, ob0, ob1, ob2, ob3,
               si0, si1, si2, si3, so0, so1, so2, so3):
        wid = lax.axis_index("s") * NC + lax.axis_index("c")

        pltpu.sync_copy(winit_hbm, wi_b)

        wi = wi_b[wid, pl.ds(0, _LANES)]
        base = wi[0]
        nch = wi[1]
        a0 = wi[2]
        ac0 = wi[3]
        poff0 = wi[4]

        obs = (ob0, ob1, ob2, ob3)
        sis = (si0, si1, si2, si3)
        sos = (so0, so1, so2, so3)

        def adv(p):
            return jnp.where(p + CH == T, 0, p + CH)

        def start_in(k, b):
            off = pl.multiple_of(base + k * CH, 8)
            pltpu.async_copy(x_hbm.at[pl.ds(off, CH)], obs[b], sis[b])

        def start_out(k, b):
            off = pl.multiple_of(base + k * CH, 8)
            pltpu.async_copy(obs[b], out_hbm.at[pl.ds(off, CH)], sos[b])

        def wait_in(b):
            pltpu.make_async_copy(x_hbm.at[pl.ds(0, CH)], obs[b], sis[b]).wait()

        def wait_out(b):
            pltpu.make_async_copy(obs[b], out_hbm.at[pl.ds(0, CH)], sos[b]).wait()

        start_in(0, 0)
        start_in(1, 1)

        pltpu.sync_copy(idx_hbm, idx_b)
        pltpu.sync_copy(ie_hbm, ie_b)
        pltpu.sync_copy(pe_hbm, pe_b)

        def ie_row_vecs(a):
            sidx = idx_b[a, pl.ds(0, _LANES)][0]
            return [ie_b[sidx, pl.ds(j * _LANES, _LANES)] for j in range(NV)]

        def fold_into_pe(vecs):
            @plsc.parallel_loop(0, T, unroll=2)
            def _(t):
                for j in range(NV):
                    plsc.addupdate(pe_b.at[t, pl.ds(j * _LANES, _LANES)], vecs[j])

        fold_into_pe(ie_row_vecs(a0))

        def compute(ob, poff):
            @plsc.parallel_loop(0, CH, unroll=2)
            def row(r):
                for j in range(NV):
                    sl = pl.ds(j * _LANES, _LANES)
                    if j % 2 == 0:
                        plsc.addupdate(ob.at[r, sl], pe_b[poff + r, sl])
                    else:
                        ob[r, sl] = ob[r, sl] + pe_b[poff + r, sl]

        def step(k, carry):
            a, ac, poff = carry

            def do(b4):
                wait_in(b4)
                compute(obs[b4], poff)
                start_out(k, b4)

                @pl.when(k > 1)
                def _():
                    wait_out((b4 + 2) % 4)

                @pl.when(k + 2 < nch)
                def _():
                    start_in(k + 2, (b4 + 2) % 4)

            km = jnp.bitwise_and(k, 3)
            for b4 in range(4):
                @pl.when(km == b4)
                def _(b4=b4):
                    do(b4)

            poff = adv(poff)
            ac = ac + 1
            bump = ac == CPA

            @pl.when(bump)
            def _():
                old = ie_row_vecs(a)
                new = ie_row_vecs(a + 1)
                fold_into_pe([n - o for n, o in zip(new, old)])

            a = jnp.where(bump, a + 1, a)
            ac = jnp.where(bump, 0, ac)
            return (a, ac, poff)

        lax.fori_loop(0, nch, step, (a0, ac0, poff0))

        nm = jnp.bitwise_and(nch, 3)
        for m in range(4):
            @pl.when(nm == m)
            def _(m=m):
                wait_out((m + 2) % 4)
                wait_out((m + 3) % 4)

    return sc_add


def _worker_init(R, T, P, NW=32, CH=40):
    nchunks = R // CH
    cpb = T // CH
    cpa = (P * T) // CH
    rows = []
    g0 = 0
    for w in range(NW):
        nc = nchunks // NW + (1 if w < nchunks % NW else 0)
        rows.append([g0 * CH, nc, g0 // cpa, g0 % cpa, (g0 % cpb) * CH] + [0] * 11)
        g0 += nc
    return np.asarray(rows, dtype=np.int32)


def kernel(x, pe, ie, id_enc_shuffle, num_a, num_p, num_t, t_offset):
    B, N, D = x.shape
    A = id_enc_shuffle.shape[0]
    T = pe.shape[1]
    P = N // (A * T)
    AP = 32

    x2 = x.reshape(N, D)
    pe2 = pe.reshape(T, D)
    ie2 = jnp.zeros((AP, D), jnp.float32).at[: ie.shape[1]].set(ie.reshape(ie.shape[1], D))
    idxp = jnp.zeros((AP, 16), jnp.int32).at[:A].set(
        jnp.broadcast_to(id_enc_shuffle.astype(jnp.int32)[:, None], (A, 16)))
    winit = jnp.asarray(_worker_init(N, T, P))

    out2 = _build_sc_call(N, D, T, A, P, AP)(x2, pe2, ie2, idxp, winit)
    return out2.reshape(B, N, D)


# speedup vs baseline: 1.1792x; 1.1792x over previous
import functools

import jax
import jax.numpy as jnp
import numpy as np
from jax import lax
from jax.experimental import pallas as pl
from jax.experimental.pallas import tpu as pltpu
from jax.experimental.pallas import tpu_sc as plsc

_LANES = 16


def _build_sc_call(R, D, T, A, P, AP):
    NC, NS = 2, 16
    NW = NC * NS
    CH = 40
    CPA = (P * T) // CH
    NV = D // _LANES

    assert T % CH == 0 and (P * T) % CH == 0 and CH % 8 == 0

    mesh = plsc.VectorSubcoreMesh(core_axis_name="c", subcore_axis_name="s")

    @functools.partial(
        pl.kernel,
        out_type=jax.ShapeDtypeStruct((R, D), jnp.float32),
        mesh=mesh,
        scratch_types=[
            pltpu.VMEM((T, D), jnp.float32),
            pltpu.VMEM((AP, D), jnp.float32),
            pltpu.VMEM((AP, _LANES), jnp.int32),
            pltpu.VMEM((NW, _LANES), jnp.int32),
            pltpu.VMEM((CH, D), jnp.float32),
            pltpu.VMEM((CH, D), jnp.float32),
            pltpu.VMEM((CH, D), jnp.float32),
            pltpu.VMEM((CH, D), jnp.float32),
            pltpu.SemaphoreType.DMA,
            pltpu.SemaphoreType.DMA,
            pltpu.SemaphoreType.DMA,
            pltpu.SemaphoreType.DMA,
            pltpu.SemaphoreType.DMA,
            pltpu.SemaphoreType.DMA,
            pltpu.SemaphoreType.DMA,
            pltpu.SemaphoreType.DMA,
        ],
    )
    def sc_add(x_hbm, pe_hbm, ie_hbm, idx_hbm, winit_hbm, out_hbm,
               pe_b, ie_b, idx_b, wi_b, ob0, ob1, ob2, ob3,
               si0, si1, si2, si3, so0, so1, so2, so3):
        wid = lax.axis_index("s") * NC + lax.axis_index("c")

        pltpu.sync_copy(winit_hbm, wi_b)

        wi = wi_b[wid, pl.ds(0, _LANES)]
        base = wi[0]
        nch = wi[1]
        a0 = wi[2]
        ac0 = wi[3]
        poff0 = wi[4]

        obs = (ob0, ob1, ob2, ob3)
        sis = (si0, si1, si2, si3)
        sos = (so0, so1, so2, so3)

        def adv(p):
            return jnp.where(p + CH == T, 0, p + CH)

        def start_in(k, b):
            off = pl.multiple_of(base + k * CH, 8)
            pltpu.async_copy(x_hbm.at[pl.ds(off, CH)], obs[b], sis[b])

        def start_out(k, b):
            off = pl.multiple_of(base + k * CH, 8)
            pltpu.async_copy(obs[b], out_hbm.at[pl.ds(off, CH)], sos[b])

        def wait_in(b):
            pltpu.make_async_copy(x_hbm.at[pl.ds(0, CH)], obs[b], sis[b]).wait()

        def wait_out(b):
            pltpu.make_async_copy(obs[b], out_hbm.at[pl.ds(0, CH)], sos[b]).wait()

        start_in(0, 0)
        start_in(1, 1)

        pltpu.sync_copy(idx_hbm, idx_b)
        pltpu.sync_copy(ie_hbm, ie_b)
        pltpu.sync_copy(pe_hbm, pe_b)

        def ie_row_vecs(a):
            sidx = idx_b[a, pl.ds(0, _LANES)][0]
            return [ie_b[sidx, pl.ds(j * _LANES, _LANES)] for j in range(NV)]

        def fold_into_pe(vecs):
            @plsc.parallel_loop(0, T, unroll=2)
            def _(t):
                for j in range(NV):
                    plsc.addupdate(pe_b.at[t, pl.ds(j * _LANES, _LANES)], vecs[j])

        fold_into_pe(ie_row_vecs(a0))

        def compute(ob, poff):
            @plsc.parallel_loop(0, CH, unroll=2)
            def row(r):
                for j in range(NV):
                    sl = pl.ds(j * _LANES, _LANES)
                    plsc.addupdate(ob.at[r, sl], pe_b[poff + r, sl])

        def step(k, carry):
            a, ac, poff = carry

            def do(b4):
                wait_in(b4)

                @pl.when(k > 1)
                def _():
                    wait_out((b4 + 2) % 4)

                @pl.when(k + 2 < nch)
                def _():
                    start_in(k + 2, (b4 + 2) % 4)

                compute(obs[b4], poff)
                start_out(k, b4)

            km = jnp.bitwise_and(k, 3)
            for b4 in range(4):
                @pl.when(km == b4)
                def _(b4=b4):
                    do(b4)

            poff = adv(poff)
            ac = ac + 1
            bump = ac == CPA

            @pl.when(bump)
            def _():
                old = ie_row_vecs(a)
                new = ie_row_vecs(a + 1)
                fold_into_pe([n - o for n, o in zip(new, old)])

            a = jnp.where(bump, a + 1, a)
            ac = jnp.where(bump, 0, ac)
            return (a, ac, poff)

        lax.fori_loop(0, nch, step, (a0, ac0, poff0))

        nm = jnp.bitwise_and(nch, 3)
        for m in range(4):
            @pl.when(nm == m)
            def _(m=m):
                wait_out((m + 2) % 4)
                wait_out((m + 3) % 4)

    return sc_add


def _worker_init(R, T, P, NW=32, CH=40):
    nchunks = R // CH
    cpb = T // CH
    cpa = (P * T) // CH
    rows = []
    g0 = 0
    for w in range(NW):
        nc = nchunks // NW + (1 if w < nchunks % NW else 0)
        rows.append([g0 * CH, nc, g0 // cpa, g0 % cpa, (g0 % cpb) * CH] + [0] * 11)
        g0 += nc
    return np.asarray(rows, dtype=np.int32)


def kernel(x, pe, ie, id_enc_shuffle, num_a, num_p, num_t, t_offset):
    B, N, D = x.shape
    A = id_enc_shuffle.shape[0]
    T = pe.shape[1]
    P = N // (A * T)
    AP = 32

    x2 = x.reshape(N, D)
    pe2 = pe.reshape(T, D)
    ie2 = jnp.zeros((AP, D), jnp.float32).at[: ie.shape[1]].set(ie.reshape(ie.shape[1], D))
    idxp = jnp.zeros((AP, 16), jnp.int32).at[:A].set(
        jnp.broadcast_to(id_enc_shuffle.astype(jnp.int32)[:, None], (A, 16)))
    winit = jnp.asarray(_worker_init(N, T, P))

    out2 = _build_sc_call(N, D, T, A, P, AP)(x2, pe2, ie2, idxp, winit)
    return out2.reshape(B, N, D)


# speedup vs baseline: 1.1801x; 1.0008x over previous
import functools

import jax
import jax.numpy as jnp
import numpy as np
from jax import lax
from jax.experimental import pallas as pl
from jax.experimental.pallas import tpu as pltpu
from jax.experimental.pallas import tpu_sc as plsc

_LANES = 16


def _build_sc_call(R, D, T, A, P, AP):
    NC, NS = 2, 16
    NW = NC * NS
    CH = 40
    CPA = (P * T) // CH
    NV = D // _LANES

    assert T % CH == 0 and (P * T) % CH == 0 and CH % 8 == 0

    mesh = plsc.VectorSubcoreMesh(core_axis_name="c", subcore_axis_name="s")

    @functools.partial(
        pl.kernel,
        out_type=jax.ShapeDtypeStruct((R, D), jnp.float32),
        mesh=mesh,
        scratch_types=[
            pltpu.VMEM((T, D), jnp.float32),
            pltpu.VMEM((AP, D), jnp.float32),
            pltpu.VMEM((AP, _LANES), jnp.int32),
            pltpu.VMEM((NW, _LANES), jnp.int32),
            pltpu.VMEM((CH, D), jnp.float32),
            pltpu.VMEM((CH, D), jnp.float32),
            pltpu.VMEM((CH, D), jnp.float32),
            pltpu.VMEM((CH, D), jnp.float32),
            pltpu.VMEM((CH, D), jnp.float32),
            pltpu.VMEM((CH, D), jnp.float32),
            pltpu.SemaphoreType.DMA,
            pltpu.SemaphoreType.DMA,
            pltpu.SemaphoreType.DMA,
            pltpu.SemaphoreType.DMA,
            pltpu.SemaphoreType.DMA,
            pltpu.SemaphoreType.DMA,
            pltpu.SemaphoreType.DMA,
            pltpu.SemaphoreType.DMA,
            pltpu.SemaphoreType.DMA,
            pltpu.SemaphoreType.DMA,
            pltpu.SemaphoreType.DMA,
            pltpu.SemaphoreType.DMA,
        ],
    )
    def sc_add(x_hbm, pe_hbm, ie_hbm, idx_hbm, winit_hbm, out_hbm,
               pe_b, ie_b, idx_b, wi_b, ob0, ob1, ob2, ob3, ob4, ob5,
               si0, si1, si2, si3, si4, si5,
               so0, so1, so2, so3, so4, so5):
        wid = lax.axis_index("s") * NC + lax.axis_index("c")

        pltpu.sync_copy(winit_hbm, wi_b)

        wi = wi_b[wid, pl.ds(0, _LANES)]
        base = wi[0]
        nch = wi[1]
        a0 = wi[2]
        ac0 = wi[3]
        poff0 = wi[4]

        obs = (ob0, ob1, ob2, ob3, ob4, ob5)
        sis = (si0, si1, si2, si3, si4, si5)
        sos = (so0, so1, so2, so3, so4, so5)
        NB = 6

        def adv(p):
            return jnp.where(p + CH == T, 0, p + CH)

        def start_in(k, b):
            off = pl.multiple_of(base + k * CH, 8)
            pltpu.async_copy(x_hbm.at[pl.ds(off, CH)], obs[b], sis[b])

        def start_out(k, b):
            off = pl.multiple_of(base + k * CH, 8)
            pltpu.async_copy(obs[b], out_hbm.at[pl.ds(off, CH)], sos[b])

        def wait_in(b):
            pltpu.make_async_copy(x_hbm.at[pl.ds(0, CH)], obs[b], sis[b]).wait()

        def wait_out(b):
            pltpu.make_async_copy(obs[b], out_hbm.at[pl.ds(0, CH)], sos[b]).wait()

        start_in(0, 0)
        start_in(1, 1)
        start_in(2, 2)
        start_in(3, 3)

        pltpu.sync_copy(idx_hbm, idx_b)
        pltpu.sync_copy(ie_hbm, ie_b)
        pltpu.sync_copy(pe_hbm, pe_b)

        def ie_row_vecs(a):
            sidx = idx_b[a, pl.ds(0, _LANES)][0]
            return [ie_b[sidx, pl.ds(j * _LANES, _LANES)] for j in range(NV)]

        def fold_into_pe(vecs):
            @plsc.parallel_loop(0, T, unroll=2)
            def _(t):
                for j in range(NV):
                    plsc.addupdate(pe_b.at[t, pl.ds(j * _LANES, _LANES)], vecs[j])

        fold_into_pe(ie_row_vecs(a0))

        def compute(ob, poff):
            @plsc.parallel_loop(0, CH, unroll=2)
            def row(r):
                for j in range(NV):
                    sl = pl.ds(j * _LANES, _LANES)
                    plsc.addupdate(ob.at[r, sl], pe_b[poff + r, sl])

        def step(k, carry):
            a, ac, poff, rb = carry

            def do(b):
                wait_in(b)

                @pl.when(k > 1)
                def _():
                    wait_out((b + 4) % NB)

                @pl.when(k + 4 < nch)
                def _():
                    start_in(k + 4, (b + 4) % NB)

                compute(obs[b], poff)
                start_out(k, b)

            for b in range(NB):
                @pl.when(rb == b)
                def _(b=b):
                    do(b)

            rb = jnp.where(rb == NB - 1, 0, rb + 1)
            poff = adv(poff)
            ac = ac + 1
            bump = ac == CPA

            @pl.when(bump)
            def _():
                old = ie_row_vecs(a)
                new = ie_row_vecs(a + 1)
                fold_into_pe([n - o for n, o in zip(new, old)])

            a = jnp.where(bump, a + 1, a)
            ac = jnp.where(bump, 0, ac)
            return (a, ac, poff, rb)

        fin = lax.fori_loop(0, nch, step, (a0, ac0, poff0, jnp.int32(0)))
        nm = fin[3]

        for m in range(NB):
            @pl.when(nm == m)
            def _(m=m):
                wait_out((m + NB - 2) % NB)
                wait_out((m + NB - 1) % NB)

    return sc_add


def _worker_init(R, T, P, NW=32, CH=40):
    nchunks = R // CH
    cpb = T // CH
    cpa = (P * T) // CH
    rows = []
    g0 = 0
    for w in range(NW):
        nc = nchunks // NW + (1 if w < nchunks % NW else 0)
        rows.append([g0 * CH, nc, g0 // cpa, g0 % cpa, (g0 % cpb) * CH] + [0] * 11)
        g0 += nc
    return np.asarray(rows, dtype=np.int32)


def kernel(x, pe, ie, id_enc_shuffle, num_a, num_p, num_t, t_offset):
    B, N, D = x.shape
    A = id_enc_shuffle.shape[0]
    T = pe.shape[1]
    P = N // (A * T)
    AP = 32

    x2 = x.reshape(N, D)
    pe2 = pe.reshape(T, D)
    ie2 = jnp.zeros((AP, D), jnp.float32).at[: ie.shape[1]].set(ie.reshape(ie.shape[1], D))
    idxp = jnp.zeros((AP, 16), jnp.int32).at[:A].set(
        jnp.broadcast_to(id_enc_shuffle.astype(jnp.int32)[:, None], (A, 16)))
    winit = jnp.asarray(_worker_init(N, T, P))

    out2 = _build_sc_call(N, D, T, A, P, AP)(x2, pe2, ie2, idxp, winit)
    return out2.reshape(B, N, D)


# speedup vs baseline: 1.1831x; 1.0025x over previous
import functools

import jax
import jax.numpy as jnp
import numpy as np
from jax import lax
from jax.experimental import pallas as pl
from jax.experimental.pallas import tpu as pltpu
from jax.experimental.pallas import tpu_sc as plsc

_LANES = 16


def _build_sc_call(R, D, T, A, P, AP):
    NC, NS = 2, 16
    NW = NC * NS
    CH = 40
    CPA = (P * T) // CH
    NV = D // _LANES

    assert T % CH == 0 and (P * T) % CH == 0 and CH % 8 == 0

    mesh = plsc.VectorSubcoreMesh(core_axis_name="c", subcore_axis_name="s")

    @functools.partial(
        pl.kernel,
        out_type=jax.ShapeDtypeStruct((R, D), jnp.float32),
        mesh=mesh,
        scratch_types=[
            pltpu.VMEM((T, D), jnp.float32),
            pltpu.VMEM((AP, D), jnp.float32),
            pltpu.VMEM((AP, _LANES), jnp.int32),
            pltpu.VMEM((NW, _LANES), jnp.int32),
            pltpu.VMEM((CH, D), jnp.float32),
            pltpu.VMEM((CH, D), jnp.float32),
            pltpu.VMEM((CH, D), jnp.float32),
            pltpu.VMEM((CH, D), jnp.float32),
            pltpu.SemaphoreType.DMA,
            pltpu.SemaphoreType.DMA,
            pltpu.SemaphoreType.DMA,
            pltpu.SemaphoreType.DMA,
            pltpu.SemaphoreType.DMA,
            pltpu.SemaphoreType.DMA,
            pltpu.SemaphoreType.DMA,
            pltpu.SemaphoreType.DMA,
        ],
    )
    def sc_add(x_hbm, pe_hbm, ie_hbm, idx_hbm, winit_hbm, out_hbm,
               pe_b, ie_b, idx_b, wi_b, ob0, ob1, ob2, ob3,
               si0, si1, si2, si3, so0, so1, so2, so3):
        wid = lax.axis_index("s") * NC + lax.axis_index("c")

        pltpu.sync_copy(winit_hbm, wi_b)

        wi = wi_b[wid, pl.ds(0, _LANES)]
        base = wi[0]
        nch = wi[1]
        a0 = wi[2]
        ac0 = wi[3]
        poff0 = wi[4]

        obs = (ob0, ob1, ob2, ob3)
        sis = (si0, si1, si2, si3)
        sos = (so0, so1, so2, so3)

        def adv(p):
            return jnp.where(p + CH == T, 0, p + CH)

        def start_in(k, b):
            off = pl.multiple_of(base + k * CH, 8)
            pltpu.async_copy(x_hbm.at[pl.ds(off, CH)], obs[b], sis[b])

        def start_out(k, b):
            off = pl.multiple_of(base + k * CH, 8)
            pltpu.async_copy(obs[b], out_hbm.at[pl.ds(off, CH)], sos[b])

        def wait_in(b):
            pltpu.make_async_copy(x_hbm.at[pl.ds(0, CH)], obs[b], sis[b]).wait()

        def wait_out(b):
            pltpu.make_async_copy(obs[b], out_hbm.at[pl.ds(0, CH)], sos[b]).wait()

        start_in(0, 0)
        start_in(1, 1)

        pltpu.sync_copy(idx_hbm, idx_b)
        pltpu.sync_copy(ie_hbm, ie_b)
        pltpu.sync_copy(pe_hbm, pe_b)

        def ie_row_vecs(a):
            sidx = idx_b[a, pl.ds(0, _LANES)][0]
            return [ie_b[sidx, pl.ds(j * _LANES, _LANES)] for j in range(NV)]

        def fold_into_pe(vecs):
            @plsc.parallel_loop(0, T, unroll=2)
            def _(t):
                for j in range(NV):
                    plsc.addupdate(pe_b.at[t, pl.ds(j * _LANES, _LANES)], vecs[j])

        fold_into_pe(ie_row_vecs(a0))

        def compute(ob, poff):
            @plsc.parallel_loop(0, CH, unroll=2)
            def row(r):
                for j in range(NV):
                    sl = pl.ds(j * _LANES, _LANES)
                    plsc.addupdate(ob.at[r, sl], pe_b[poff + r, sl])

        def step(k, carry):
            a, ac, poff = carry

            def do(b4):
                wait_in(b4)

                @pl.when(k > 1)
                def _():
                    wait_out((b4 + 2) % 4)

                @pl.when(k + 2 < nch)
                def _():
                    start_in(k + 2, (b4 + 2) % 4)

                compute(obs[b4], poff)
                start_out(k, b4)

            km = jnp.bitwise_and(k, 3)
            for b4 in range(4):
                @pl.when(km == b4)
                def _(b4=b4):
                    do(b4)

            poff = adv(poff)
            ac = ac + 1
            bump = ac == CPA

            @pl.when(bump)
            def _():
                old = ie_row_vecs(a)
                new = ie_row_vecs(a + 1)
                fold_into_pe([n - o for n, o in zip(new, old)])

            a = jnp.where(bump, a + 1, a)
            ac = jnp.where(bump, 0, ac)
            return (a, ac, poff)

        lax.fori_loop(0, nch, step, (a0, ac0, poff0))

        nm = jnp.bitwise_and(nch, 3)
        for m in range(4):
            @pl.when(nm == m)
            def _(m=m):
                wait_out((m + 2) % 4)
                wait_out((m + 3) % 4)

    return sc_add


def _worker_init(R, T, P, NW=32, CH=40):
    nchunks = R // CH
    cpb = T // CH
    cpa = (P * T) // CH
    rows = []
    g0 = 0
    for w in range(NW):
        nc = nchunks // NW + (1 if w < nchunks % NW else 0)
        rows.append([g0 * CH, nc, g0 // cpa, g0 % cpa, (g0 % cpb) * CH] + [0] * 11)
        g0 += nc
    return np.asarray(rows, dtype=np.int32)


def kernel(x, pe, ie, id_enc_shuffle, num_a, num_p, num_t, t_offset):
    B, N, D = x.shape
    A = id_enc_shuffle.shape[0]
    T = pe.shape[1]
    P = N // (A * T)
    AP = 32

    x2 = x.reshape(N, D)
    pe2 = pe.reshape(T, D)
    ie2 = jnp.zeros((AP, D), jnp.float32).at[: ie.shape[1]].set(ie.reshape(ie.shape[1], D))
    idxp = jnp.zeros((AP, 16), jnp.int32).at[:A].set(
        jnp.broadcast_to(id_enc_shuffle.astype(jnp.int32)[:, None], (A, 16)))
    winit = jnp.asarray(_worker_init(N, T, P))

    out2 = _build_sc_call(N, D, T, A, P, AP)(x2, pe2, ie2, idxp, winit)
    return out2.reshape(B, N, D)
